# Initial kernel scaffold; baseline (speedup 1.0000x reference)
import jax
import jax.numpy as jnp
from jax.experimental import pallas as pl


def kernel(child_feats, child_exists, edge_type_onehot, edge_indices, W_child, b_child, W_ne0, b_ne0, W_ne1, b_ne1, W_parent, b_parent):
    raise NotImplementedError("write your pallas kernel here")


# speedup vs baseline: 2.6032x; 2.6032x over previous
import functools

import jax
import jax.numpy as jnp
from jax import lax
from jax.experimental import pallas as pl
from jax.experimental.pallas import tpu as pltpu
from jax.experimental.pallas import tpu_sc as plsc

_N = 10000
_E = 320000
_D = 128
_H = 128
_ET = 4

_NC = 2
_NS = 16
_L = 16
_NW = _NC * _NS
_EPW = _E // _NW
_K = 80
_NCH = _EPW // _K
_NP = 10240
_RPT = _NP // _NS
_ZR = 128
_BLK = 2048


def _sc_edge_pass(A, B, ef_flat, src, dst, wcb):
    mesh = plsc.VectorSubcoreMesh(core_axis_name="c", subcore_axis_name="s")

    @functools.partial(
        pl.kernel,
        mesh=mesh,
        out_type=jax.ShapeDtypeStruct((_NC, _NP, _H), jnp.float32),
        scratch_types=[
            pltpu.VMEM((_K,), jnp.int32),
            pltpu.VMEM((_K,), jnp.int32),
            pltpu.VMEM((_K * _ET + _L,), jnp.float32),
            pltpu.VMEM((_K, _H), jnp.float32),
            pltpu.VMEM((_K, _H), jnp.float32),
            pltpu.VMEM((_ZR, _H), jnp.float32),
            pltpu.VMEM((_ET + 1, _H), jnp.float32),
            pltpu.VMEM_SHARED((_NP, _H), jnp.float32),
            pltpu.SemaphoreType.DMA,
            pltpu.SemaphoreType.DMA,
        ],
    )
    def k(a_h, b_h, ef_h, src_h, dst_h, wcb_h, out_h,
          sidx, didx, efb, bufa, bufb, zbuf, wbuf, acc, sema, semb):
        cid = lax.axis_index("c")
        sid = lax.axis_index("s")
        wid = sid * _NC + cid

        pltpu.sync_copy(wcb_h, wbuf)
        wregs = [[wbuf[t, pl.ds(j * _L, _L)] for j in range(_H // _L)]
                 for t in range(_ET + 1)]

        def zrow(r, carry):
            for j in range(_H // _L):
                zbuf[r, pl.ds(j * _L, _L)] = jnp.zeros((_L,), jnp.float32)
            return carry

        lax.fori_loop(0, _ZR, zrow, 0)
        for i in range(_RPT // _ZR):
            pltpu.sync_copy(zbuf, acc.at[pl.ds(sid * _RPT + i * _ZR, _ZR)])
        plsc.subcore_barrier()

        base = wid * _EPW

        def chunk(c, carry):
            off = base + c * _K
            pltpu.sync_copy(src_h.at[pl.ds(off, _K)], sidx)
            pltpu.sync_copy(dst_h.at[pl.ds(off, _K)], didx)
            pltpu.sync_copy(ef_h.at[pl.ds(off * _ET, _K * _ET)],
                            efb.at[pl.ds(0, _K * _ET)])
            cpa = pltpu.async_copy(a_h.at[sidx], bufa, sema)
            cpb = pltpu.async_copy(b_h.at[didx], bufb, semb)
            cpa.wait()
            cpb.wait()

            def row(e, rcarry):
                eb = e * _ET
                efv = efb[pl.ds(eb, _L)]
                ef0 = jnp.full((_L,), efv[0], jnp.float32)
                ef1 = jnp.full((_L,), efv[1], jnp.float32)
                ef2 = jnp.full((_L,), efv[2], jnp.float32)
                ef3 = jnp.full((_L,), efv[3], jnp.float32)
                for j in range(_H // _L):
                    sl = pl.ds(j * _L, _L)
                    v = bufa[e, sl] + bufb[e, sl] + wregs[_ET][j]
                    v = v + ef0 * wregs[0][j] + ef1 * wregs[1][j]
                    v = v + ef2 * wregs[2][j] + ef3 * wregs[3][j]
                    bufa[e, sl] = jnp.maximum(v, 0.0)
                return rcarry

            lax.fori_loop(0, _K, row, 0)
            pltpu.sync_copy(bufa, acc.at[sidx], add=True)
            return carry

        lax.fori_loop(0, _NCH, chunk, 0)
        plsc.subcore_barrier()
        pltpu.sync_copy(acc.at[pl.ds(sid * _RPT, _RPT)],
                        out_h.at[cid, pl.ds(sid * _RPT, _RPT)])

    return k(A, B, ef_flat, src, dst, wcb)


def _tc_prep(x, w_child, b_child, wa, wb):
    def body(x_ref, wc_ref, bc_ref, wa_ref, wb_ref, a_ref, b_ref, p_ref):
        i = pl.program_id(0)
        cf = jnp.maximum(
            jnp.dot(x_ref[...], wc_ref[...],
                    preferred_element_type=jnp.float32) + bc_ref[...], 0.0)
        a_ref[...] = jnp.dot(cf, wa_ref[...], preferred_element_type=jnp.float32)
        b_ref[...] = jnp.dot(cf, wb_ref[...], preferred_element_type=jnp.float32)
        s = jnp.sum(cf, axis=0, keepdims=True)

        @pl.when(i == 0)
        def _():
            p_ref[...] = s

        @pl.when(i != 0)
        def _():
            p_ref[...] = p_ref[...] + s

    full = pl.BlockSpec((_H, _H), lambda i: (0, 0))
    row1 = pl.BlockSpec((1, _H), lambda i: (0, 0))
    nblk = pl.BlockSpec((_BLK, _H), lambda i: (i, 0))
    return pl.pallas_call(
        body,
        grid=(_NP // _BLK,),
        in_specs=[nblk, full, row1, full, full],
        out_specs=[nblk, nblk, row1],
        out_shape=[
            jax.ShapeDtypeStruct((_NP, _H), jnp.float32),
            jax.ShapeDtypeStruct((_NP, _H), jnp.float32),
            jax.ShapeDtypeStruct((1, _H), jnp.float32),
        ],
    )(x, w_child, b_child, wa, wb)


def _tc_mid(parts, wa, wb):
    def body(p_ref, wa_ref, wb_ref, a_ref, b_ref, s_ref):
        i = pl.program_id(0)
        cf = p_ref[0] + p_ref[1]
        a_ref[...] = jnp.dot(cf, wa_ref[...], preferred_element_type=jnp.float32)
        b_ref[...] = jnp.dot(cf, wb_ref[...], preferred_element_type=jnp.float32)
        s = jnp.sum(cf, axis=0, keepdims=True)

        @pl.when(i == 0)
        def _():
            s_ref[...] = s

        @pl.when(i != 0)
        def _():
            s_ref[...] = s_ref[...] + s

    full = pl.BlockSpec((_H, _H), lambda i: (0, 0))
    row1 = pl.BlockSpec((1, _H), lambda i: (0, 0))
    nblk = pl.BlockSpec((_BLK, _H), lambda i: (i, 0))
    pblk = pl.BlockSpec((2, _BLK, _H), lambda i: (0, i, 0))
    return pl.pallas_call(
        body,
        grid=(_NP // _BLK,),
        in_specs=[pblk, full, full],
        out_specs=[nblk, nblk, row1],
        out_shape=[
            jax.ShapeDtypeStruct((_NP, _H), jnp.float32),
            jax.ShapeDtypeStruct((_NP, _H), jnp.float32),
            jax.ShapeDtypeStruct((1, _H), jnp.float32),
        ],
    )(parts, wa, wb)


def _tc_fin(parts, p0, p1, wp0, wp1, wp2, bp):
    def body(parts_ref, p0_ref, p1_ref, w0_ref, w1_ref, w2_ref, bp_ref,
             out_ref, acc_ref):
        i = pl.program_id(0)
        s = jnp.sum(parts_ref[0] + parts_ref[1], axis=0, keepdims=True)

        @pl.when(i == 0)
        def _():
            acc_ref[...] = s

        @pl.when(i != 0)
        def _():
            acc_ref[...] = acc_ref[...] + s

        @pl.when(i == pl.num_programs(0) - 1)
        def _():
            r = jnp.dot(p0_ref[...], w0_ref[...],
                        preferred_element_type=jnp.float32)
            r = r + jnp.dot(p1_ref[...], w1_ref[...],
                            preferred_element_type=jnp.float32)
            r = r + jnp.dot(acc_ref[...], w2_ref[...],
                            preferred_element_type=jnp.float32)
            out_ref[...] = jnp.maximum(r + bp_ref[...], 0.0)

    full = pl.BlockSpec((_H, _H), lambda i: (0, 0))
    row1 = pl.BlockSpec((1, _H), lambda i: (0, 0))
    pblk = pl.BlockSpec((2, _BLK, _H), lambda i: (0, i, 0))
    return pl.pallas_call(
        body,
        grid=(_NP // _BLK,),
        in_specs=[pblk, row1, row1, full, full, full, row1],
        out_specs=row1,
        out_shape=jax.ShapeDtypeStruct((1, _D), jnp.float32),
        scratch_shapes=[pltpu.VMEM((1, _H), jnp.float32)],
    )(parts, p0, p1, wp0, wp1, wp2, bp)


def kernel(child_feats, child_exists, edge_type_onehot, edge_indices,
           W_child, b_child, W_ne0, b_ne0, W_ne1, b_ne1, W_parent, b_parent):
    x = (child_feats * child_exists)[0]
    x = jnp.concatenate([x, jnp.zeros((_NP - _N, _D), jnp.float32)], axis=0)
    src = edge_indices[0, :, 0]
    dst = edge_indices[0, :, 1]
    ef_flat = edge_type_onehot.reshape(_E * _ET)

    wa0, wb0 = W_ne0[:_H], W_ne0[_H:2 * _H]
    wa1, wb1 = W_ne1[:_H], W_ne1[_H:2 * _H]
    wcb0 = jnp.concatenate([W_ne0[2 * _H:], b_ne0[None, :]], axis=0)
    wcb1 = jnp.concatenate([W_ne1[2 * _H:], b_ne1[None, :]], axis=0)

    a0, b0, p0 = _tc_prep(x, W_child, b_child.reshape(1, _H), wa0, wb0)
    parts1 = _sc_edge_pass(a0, b0, ef_flat, src, dst, wcb0)
    a1, b1, p1 = _tc_mid(parts1, wa1, wb1)
    parts2 = _sc_edge_pass(a1, b1, ef_flat, src, dst, wcb1)
    return _tc_fin(parts2, p0, p1,
                   W_parent[:_H], W_parent[_H:2 * _H], W_parent[2 * _H:],
                   b_parent.reshape(1, _D))
